# Initial kernel scaffold; baseline (speedup 1.0000x reference)
import functools

import jax
import jax.numpy as jnp
from jax import lax
from jax.experimental import pallas as pl
from jax.experimental.pallas import tpu as pltpu
from jax.experimental.pallas import tpu_sc as plsc

N = 10000
E = 320000
IN_DIM = 128
HID = 64
OUT_DIM = 128

NC = 2
NS = 16
CHUNK = 80
EDGES_PER_TILE = E // (NC * NS)
STEPS = EDGES_PER_TILE // CHUNK
ROWS_PER_TILE = N // NS

_MESH = plsc.VectorSubcoreMesh(core_axis_name="c", subcore_axis_name="s")


@functools.partial(
    pl.kernel,
    mesh=_MESH,
    out_type=jax.ShapeDtypeStruct((NC, N), jnp.float32),
    scratch_types=[
        pltpu.VMEM((CHUNK,), jnp.int32),
        pltpu.VMEM((CHUNK,), jnp.float32),
        pltpu.VMEM_SHARED((N,), jnp.float32),
    ],
)
def _deg_partials(ones_hbm, dst_hbm, out_hbm, idx_v, ones_v, deg_sh):
    c = lax.axis_index("c")
    s = lax.axis_index("s")
    r0 = s * ROWS_PER_TILE
    pltpu.sync_copy(ones_hbm.at[pl.ds(r0, ROWS_PER_TILE)],
                    deg_sh.at[pl.ds(r0, ROWS_PER_TILE)])
    for i in range(CHUNK // 16):
        ones_v[pl.ds(i * 16, 16)] = jnp.ones((16,), jnp.float32)
    plsc.subcore_barrier()
    tile_base = (c * NS + s) * EDGES_PER_TILE

    def body(i, carry):
        b = tile_base + i * CHUNK
        pltpu.sync_copy(dst_hbm.at[pl.ds(b, CHUNK)], idx_v)
        pltpu.sync_copy(ones_v, deg_sh.at[idx_v], add=True)
        return carry

    lax.fori_loop(0, STEPS, body, 0)
    plsc.subcore_barrier()
    pltpu.sync_copy(deg_sh.at[pl.ds(r0, ROWS_PER_TILE)],
                    out_hbm.at[c, pl.ds(r0, ROWS_PER_TILE)])


@functools.partial(
    pl.kernel,
    mesh=_MESH,
    out_type=jax.ShapeDtypeStruct((NC, N, HID), jnp.float32),
    scratch_types=[
        pltpu.VMEM((CHUNK,), jnp.int32),
        pltpu.VMEM((CHUNK,), jnp.int32),
        pltpu.VMEM((CHUNK, HID), jnp.float32),
        pltpu.VMEM_SHARED((N, HID), jnp.float32),
        pltpu.VMEM_SHARED((N, HID), jnp.float32),
    ],
)
def _agg_partials(h_hbm, src_hbm, dst_hbm, out_hbm,
                  sidx_v, didx_v, rows_v, h_sh, acc_sh):
    c = lax.axis_index("c")
    s = lax.axis_index("s")
    r0 = s * ROWS_PER_TILE
    pltpu.sync_copy(h_hbm.at[pl.ds(r0, ROWS_PER_TILE)],
                    h_sh.at[pl.ds(r0, ROWS_PER_TILE)])
    pltpu.sync_copy(h_hbm.at[pl.ds(r0, ROWS_PER_TILE)],
                    acc_sh.at[pl.ds(r0, ROWS_PER_TILE)])
    plsc.subcore_barrier()
    tile_base = (c * NS + s) * EDGES_PER_TILE

    def body(i, carry):
        b = tile_base + i * CHUNK
        pltpu.sync_copy(src_hbm.at[pl.ds(b, CHUNK)], sidx_v)
        pltpu.sync_copy(dst_hbm.at[pl.ds(b, CHUNK)], didx_v)
        pltpu.sync_copy(h_sh.at[sidx_v], rows_v)
        pltpu.sync_copy(rows_v, acc_sh.at[didx_v], add=True)
        return carry

    lax.fori_loop(0, STEPS, body, 0)
    plsc.subcore_barrier()
    pltpu.sync_copy(acc_sh.at[pl.ds(r0, ROWS_PER_TILE)],
                    out_hbm.at[c, pl.ds(r0, ROWS_PER_TILE)])


BLK = 1000


def _dinv(dp_ref):
    deg = dp_ref[:, 0:1] + dp_ref[:, 1:2] - 1.0
    return lax.rsqrt(deg)


def _tc_in_body(x_ref, w_ref, dp_ref, o_ref):
    o_ref[...] = jnp.dot(x_ref[...], w_ref[...],
                         preferred_element_type=jnp.float32) * _dinv(dp_ref)


def _tc_mid_body(a0_ref, a1_ref, h_ref, dp_ref, b_ref, o_ref):
    dinv = _dinv(dp_ref)
    agg = a0_ref[...] + a1_ref[...] - h_ref[...]
    pre = agg * dinv + b_ref[...]
    o_ref[...] = jnp.maximum(pre, 0.0) * dinv


def _tc_out_body(a0_ref, a1_ref, h_ref, dp_ref, w_ref, b_ref, o_ref):
    dinv = _dinv(dp_ref)
    agg = (a0_ref[...] + a1_ref[...] - h_ref[...]) * dinv
    z = jnp.dot(agg, w_ref[...], preferred_element_type=jnp.float32) + b_ref[...]
    m = jnp.max(z, axis=1, keepdims=True)
    lse = jnp.log(jnp.sum(jnp.exp(z - m), axis=1, keepdims=True))
    o_ref[...] = z - m - lse


_tc_in = pl.pallas_call(
    _tc_in_body,
    grid=(N // BLK,),
    in_specs=[
        pl.BlockSpec((BLK, IN_DIM), lambda i: (i, 0)),
        pl.BlockSpec((IN_DIM, HID), lambda i: (0, 0)),
        pl.BlockSpec((BLK, 2), lambda i: (i, 0)),
    ],
    out_specs=pl.BlockSpec((BLK, HID), lambda i: (i, 0)),
    out_shape=jax.ShapeDtypeStruct((N, HID), jnp.float32),
)

_tc_mid = pl.pallas_call(
    _tc_mid_body,
    grid=(N // BLK,),
    in_specs=[
        pl.BlockSpec((BLK, HID), lambda i: (i, 0)),
        pl.BlockSpec((BLK, HID), lambda i: (i, 0)),
        pl.BlockSpec((BLK, HID), lambda i: (i, 0)),
        pl.BlockSpec((BLK, 2), lambda i: (i, 0)),
        pl.BlockSpec((1, HID), lambda i: (0, 0)),
    ],
    out_specs=pl.BlockSpec((BLK, HID), lambda i: (i, 0)),
    out_shape=jax.ShapeDtypeStruct((N, HID), jnp.float32),
)

_tc_out = pl.pallas_call(
    _tc_out_body,
    grid=(N // BLK,),
    in_specs=[
        pl.BlockSpec((BLK, HID), lambda i: (i, 0)),
        pl.BlockSpec((BLK, HID), lambda i: (i, 0)),
        pl.BlockSpec((BLK, HID), lambda i: (i, 0)),
        pl.BlockSpec((BLK, 2), lambda i: (i, 0)),
        pl.BlockSpec((HID, OUT_DIM), lambda i: (0, 0)),
        pl.BlockSpec((1, OUT_DIM), lambda i: (0, 0)),
    ],
    out_specs=pl.BlockSpec((BLK, OUT_DIM), lambda i: (i, 0)),
    out_shape=jax.ShapeDtypeStruct((N, OUT_DIM), jnp.float32),
)


@jax.jit
def kernel(x, edge_index, W1, b1, W2, b2):
    src = edge_index[0]
    dst = edge_index[1]
    ones = jnp.ones((N,), jnp.float32)

    degp = _deg_partials(ones, dst)
    dp = degp.T
    h1 = _tc_in(x, W1, dp)
    accp1 = _agg_partials(h1, src, dst)
    h2 = _tc_mid(accp1[0], accp1[1], h1, dp, b1.reshape(1, HID))
    accp2 = _agg_partials(h2, src, dst)
    return _tc_out(accp2[0], accp2[1], h2, dp, W2, b2.reshape(1, OUT_DIM))


# speedup vs baseline: 14.6268x; 14.6268x over previous
import functools

import jax
import jax.numpy as jnp
from jax import lax
from jax.experimental import pallas as pl
from jax.experimental.pallas import tpu as pltpu
from jax.experimental.pallas import tpu_sc as plsc

N = 10000
E = 320000
IN_DIM = 128
HID = 64
OUT_DIM = 128

NC = 2
NS = 16
CHUNK = 80
EDGES_PER_TILE = E // (NC * NS)
STEPS = EDGES_PER_TILE // CHUNK
RCHUNK = 400
NRCH = N // RCHUNK

_MESH = plsc.VectorSubcoreMesh(core_axis_name="c", subcore_axis_name="s")


@functools.partial(
    pl.kernel,
    mesh=_MESH,
    out_type=jax.ShapeDtypeStruct((NC * N,), jnp.float32),
    scratch_types=[
        pltpu.VMEM((CHUNK,), jnp.int32),
        pltpu.VMEM((CHUNK,), jnp.float32),
        pltpu.VMEM((RCHUNK,), jnp.float32),
        pltpu.VMEM_SHARED((N,), jnp.float32),
    ],
)
def _deg_partials(dst_hbm, out_hbm, idx_v, ones_v, stage_v, deg_sh):
    c = lax.axis_index("c")
    s = lax.axis_index("s")
    for i in range(CHUNK // 16):
        ones_v[pl.ds(i * 16, 16)] = jnp.ones((16,), jnp.float32)
    for i in range(RCHUNK // 16):
        stage_v[pl.ds(i * 16, 16)] = jnp.ones((16,), jnp.float32)
    for rep in range(2):
        ck = s + NS * rep

        @pl.when(ck < NRCH)
        def _():
            r0 = ck * RCHUNK
            pltpu.sync_copy(stage_v, deg_sh.at[pl.ds(r0, RCHUNK)])

    plsc.subcore_barrier()
    tile_base = (c * NS + s) * EDGES_PER_TILE

    def body(i, carry):
        b = tile_base + i * CHUNK
        pltpu.sync_copy(dst_hbm.at[pl.ds(b, CHUNK)], idx_v)
        pltpu.sync_copy(ones_v, deg_sh.at[idx_v], add=True)
        return carry

    lax.fori_loop(0, STEPS, body, 0)
    plsc.subcore_barrier()
    for rep in range(2):
        ck = s + NS * rep

        @pl.when(ck < NRCH)
        def _():
            r0 = ck * RCHUNK
            pltpu.sync_copy(deg_sh.at[pl.ds(r0, RCHUNK)], stage_v)
            pltpu.sync_copy(stage_v, out_hbm.at[pl.ds(c * N + r0, RCHUNK)])


@functools.partial(
    pl.kernel,
    mesh=_MESH,
    compiler_params=pltpu.CompilerParams(use_tc_tiling_on_sc=False),
    out_type=jax.ShapeDtypeStruct((NC, N, HID), jnp.float32),
    scratch_types=[
        pltpu.VMEM((CHUNK,), jnp.int32),
        pltpu.VMEM((CHUNK,), jnp.int32),
        pltpu.VMEM((CHUNK, HID), jnp.float32),
        pltpu.VMEM((RCHUNK, HID), jnp.float32),
        pltpu.VMEM_SHARED((N, HID), jnp.float32),
    ],
)
def _agg_partials(h_hbm, src_hbm, dst_hbm, out_hbm,
                  sidx_v, didx_v, rows_v, stage_v, acc_sh):
    c = lax.axis_index("c")
    s = lax.axis_index("s")
    for rep in range(2):
        ck = s + NS * rep

        @pl.when(ck < NRCH)
        def _():
            r0 = ck * RCHUNK
            pltpu.sync_copy(h_hbm.at[pl.ds(r0, RCHUNK)], stage_v)
            pltpu.sync_copy(stage_v, acc_sh.at[pl.ds(r0, RCHUNK)])

    plsc.subcore_barrier()
    tile_base = (c * NS + s) * EDGES_PER_TILE

    def body(i, carry):
        b = tile_base + i * CHUNK
        pltpu.sync_copy(src_hbm.at[pl.ds(b, CHUNK)], sidx_v)
        pltpu.sync_copy(dst_hbm.at[pl.ds(b, CHUNK)], didx_v)
        pltpu.sync_copy(h_hbm.at[sidx_v], rows_v)
        pltpu.sync_copy(rows_v, acc_sh.at[didx_v], add=True)
        return carry

    lax.fori_loop(0, STEPS, body, 0)
    plsc.subcore_barrier()
    for rep in range(2):
        ck = s + NS * rep

        @pl.when(ck < NRCH)
        def _():
            r0 = ck * RCHUNK
            pltpu.sync_copy(acc_sh.at[pl.ds(r0, RCHUNK)], stage_v)
            pltpu.sync_copy(stage_v, out_hbm.at[c, pl.ds(r0, RCHUNK)])


BLK = 1000


def _dinv(dp_ref):
    deg = dp_ref[:, 0:1] + dp_ref[:, 1:2] - 1.0
    return lax.rsqrt(deg)


def _tc_in_body(x_ref, w_ref, dp_ref, o_ref):
    o_ref[...] = jnp.dot(x_ref[...], w_ref[...],
                         preferred_element_type=jnp.float32) * _dinv(dp_ref)


def _tc_mid_body(a0_ref, a1_ref, h_ref, dp_ref, b_ref, o_ref):
    dinv = _dinv(dp_ref)
    agg = a0_ref[...] + a1_ref[...] - h_ref[...]
    pre = agg * dinv + b_ref[...]
    o_ref[...] = jnp.maximum(pre, 0.0) * dinv


def _tc_out_body(a0_ref, a1_ref, h_ref, dp_ref, w_ref, b_ref, o_ref):
    dinv = _dinv(dp_ref)
    agg = (a0_ref[...] + a1_ref[...] - h_ref[...]) * dinv
    z = jnp.dot(agg, w_ref[...], preferred_element_type=jnp.float32) + b_ref[...]
    m = jnp.max(z, axis=1, keepdims=True)
    lse = jnp.log(jnp.sum(jnp.exp(z - m), axis=1, keepdims=True))
    o_ref[...] = z - m - lse


_tc_in = pl.pallas_call(
    _tc_in_body,
    grid=(N // BLK,),
    in_specs=[
        pl.BlockSpec((BLK, IN_DIM), lambda i: (i, 0)),
        pl.BlockSpec((IN_DIM, HID), lambda i: (0, 0)),
        pl.BlockSpec((BLK, 2), lambda i: (i, 0)),
    ],
    out_specs=pl.BlockSpec((BLK, HID), lambda i: (i, 0)),
    out_shape=jax.ShapeDtypeStruct((N, HID), jnp.float32),
)

_tc_mid = pl.pallas_call(
    _tc_mid_body,
    grid=(N // BLK,),
    in_specs=[
        pl.BlockSpec((BLK, HID), lambda i: (i, 0)),
        pl.BlockSpec((BLK, HID), lambda i: (i, 0)),
        pl.BlockSpec((BLK, HID), lambda i: (i, 0)),
        pl.BlockSpec((BLK, 2), lambda i: (i, 0)),
        pl.BlockSpec((1, HID), lambda i: (0, 0)),
    ],
    out_specs=pl.BlockSpec((BLK, HID), lambda i: (i, 0)),
    out_shape=jax.ShapeDtypeStruct((N, HID), jnp.float32),
)

_tc_out = pl.pallas_call(
    _tc_out_body,
    grid=(N // BLK,),
    in_specs=[
        pl.BlockSpec((BLK, HID), lambda i: (i, 0)),
        pl.BlockSpec((BLK, HID), lambda i: (i, 0)),
        pl.BlockSpec((BLK, HID), lambda i: (i, 0)),
        pl.BlockSpec((BLK, 2), lambda i: (i, 0)),
        pl.BlockSpec((HID, OUT_DIM), lambda i: (0, 0)),
        pl.BlockSpec((1, OUT_DIM), lambda i: (0, 0)),
    ],
    out_specs=pl.BlockSpec((BLK, OUT_DIM), lambda i: (i, 0)),
    out_shape=jax.ShapeDtypeStruct((N, OUT_DIM), jnp.float32),
)


@jax.jit
def kernel(x, edge_index, W1, b1, W2, b2):
    src = edge_index[0]
    dst = edge_index[1]

    degp = _deg_partials(dst).reshape(NC, N)
    dp = degp.T
    h1 = _tc_in(x, W1, dp)
    accp1 = _agg_partials(h1, src, dst)
    h2 = _tc_mid(accp1[0], accp1[1], h1, dp, b1.reshape(1, HID))
    accp2 = _agg_partials(h2, src, dst)
    return _tc_out(accp2[0], accp2[1], h2, dp, W2, b2.reshape(1, OUT_DIM))


# speedup vs baseline: 42.1867x; 2.8842x over previous
import functools

import jax
import jax.numpy as jnp
from jax import lax
from jax.experimental import pallas as pl
from jax.experimental.pallas import tpu as pltpu
from jax.experimental.pallas import tpu_sc as plsc

N = 10000
E = 320000
IN_DIM = 128
HID = 64
OUT_DIM = 128

NC = 2
NS = 16
CHUNK = 80
EDGES_PER_TILE = E // (NC * NS)
STEPS = EDGES_PER_TILE // CHUNK
RCHUNK = 400
NRCH = N // RCHUNK

_MESH = plsc.VectorSubcoreMesh(core_axis_name="c", subcore_axis_name="s")


@functools.partial(
    pl.kernel,
    mesh=_MESH,
    compiler_params=pltpu.CompilerParams(use_tc_tiling_on_sc=False),
    out_type=jax.ShapeDtypeStruct((NC * N,), jnp.float32),
    scratch_types=[
        pltpu.VMEM((STEPS, CHUNK), jnp.int32),
        pltpu.VMEM((CHUNK,), jnp.float32),
        pltpu.VMEM((RCHUNK,), jnp.float32),
        pltpu.VMEM_SHARED((N,), jnp.float32),
    ],
)
def _deg_partials(dst_hbm, out_hbm, idx_v, ones_v, stage_v, deg_sh):
    c = lax.axis_index("c")
    s = lax.axis_index("s")
    tile_row = (c * NS + s) * STEPS
    pltpu.sync_copy(dst_hbm.at[pl.ds(tile_row, STEPS)], idx_v)
    for i in range(CHUNK // 16):
        ones_v[pl.ds(i * 16, 16)] = jnp.ones((16,), jnp.float32)
    for i in range(RCHUNK // 16):
        stage_v[pl.ds(i * 16, 16)] = jnp.ones((16,), jnp.float32)
    for rep in range(2):
        ck = s + NS * rep

        @pl.when(ck < NRCH)
        def _():
            r0 = ck * RCHUNK
            pltpu.sync_copy(stage_v, deg_sh.at[pl.ds(r0, RCHUNK)])

    plsc.subcore_barrier()

    def body(i, carry):
        pltpu.sync_copy(ones_v, deg_sh.at[idx_v.at[i]], add=True)
        return carry

    lax.fori_loop(0, STEPS, body, 0)
    plsc.subcore_barrier()
    for rep in range(2):
        ck = s + NS * rep

        @pl.when(ck < NRCH)
        def _():
            r0 = ck * RCHUNK
            pltpu.sync_copy(deg_sh.at[pl.ds(r0, RCHUNK)], stage_v)
            pltpu.sync_copy(stage_v, out_hbm.at[pl.ds(c * N + r0, RCHUNK)])


@functools.partial(
    pl.kernel,
    mesh=_MESH,
    compiler_params=pltpu.CompilerParams(use_tc_tiling_on_sc=False),
    out_type=jax.ShapeDtypeStruct((NC, N, HID), jnp.float32),
    scratch_types=[
        pltpu.VMEM((STEPS, CHUNK), jnp.int32),
        pltpu.VMEM((STEPS, CHUNK), jnp.int32),
        pltpu.VMEM((CHUNK, HID), jnp.float32),
        pltpu.VMEM((CHUNK, HID), jnp.float32),
        pltpu.VMEM((CHUNK, HID), jnp.float32),
        pltpu.VMEM((CHUNK, HID), jnp.float32),
        pltpu.VMEM((CHUNK, HID), jnp.float32),
        pltpu.VMEM((RCHUNK, HID), jnp.float32),
        pltpu.VMEM_SHARED((N, HID), jnp.float32),
        pltpu.SemaphoreType.DMA,
        pltpu.SemaphoreType.DMA,
        pltpu.SemaphoreType.DMA,
        pltpu.SemaphoreType.DMA,
        pltpu.SemaphoreType.DMA,
    ],
)
def _agg_partials(h_hbm, src_hbm, dst_hbm, out_hbm,
                  sidx_v, didx_v, r0_v, r1_v, r2_v, r3_v, r4_v,
                  stage_v, acc_sh, sem0, sem1, sem2, sem3, sem4):
    rows = (r0_v, r1_v, r2_v, r3_v, r4_v)
    sems = (sem0, sem1, sem2, sem3, sem4)
    NBUF = 5
    c = lax.axis_index("c")
    s = lax.axis_index("s")
    for rep in range(2):
        ck = s + NS * rep

        @pl.when(ck < NRCH)
        def _():
            r0 = ck * RCHUNK
            pltpu.sync_copy(h_hbm.at[pl.ds(r0, RCHUNK)], stage_v)
            pltpu.sync_copy(stage_v, acc_sh.at[pl.ds(r0, RCHUNK)])

    tile_row = (c * NS + s) * STEPS
    pltpu.sync_copy(src_hbm.at[pl.ds(tile_row, STEPS)], sidx_v)
    pltpu.sync_copy(dst_hbm.at[pl.ds(tile_row, STEPS)], didx_v)
    plsc.subcore_barrier()

    for b in range(NBUF):
        pltpu.async_copy(h_hbm.at[sidx_v.at[b]], rows[b], sems[b])

    def body(g, carry):
        j0 = g * NBUF
        for b in range(NBUF):
            jj = j0 + b
            pltpu.make_async_copy(h_hbm.at[sidx_v.at[b]], rows[b],
                                  sems[b]).wait()
            pltpu.sync_copy(rows[b], acc_sh.at[didx_v.at[jj]], add=True)

            @pl.when(jj + NBUF < STEPS)
            def _():
                pltpu.async_copy(h_hbm.at[sidx_v.at[jj + NBUF]],
                                 rows[b], sems[b])

        return carry

    lax.fori_loop(0, STEPS // NBUF, body, 0)
    plsc.subcore_barrier()
    for rep in range(2):
        ck = s + NS * rep

        @pl.when(ck < NRCH)
        def _():
            r0 = ck * RCHUNK
            pltpu.sync_copy(acc_sh.at[pl.ds(r0, RCHUNK)], stage_v)
            pltpu.sync_copy(stage_v, out_hbm.at[c, pl.ds(r0, RCHUNK)])


BLK = 1000


def _dinv(dp_ref):
    deg = dp_ref[:, 0:1] + dp_ref[:, 1:2] - 1.0
    return lax.rsqrt(deg)


def _tc_in_body(x_ref, w_ref, dp_ref, o_ref):
    o_ref[...] = jnp.dot(x_ref[...], w_ref[...],
                         preferred_element_type=jnp.float32) * _dinv(dp_ref)


def _tc_mid_body(a0_ref, a1_ref, h_ref, dp_ref, b_ref, o_ref):
    dinv = _dinv(dp_ref)
    agg = a0_ref[...] + a1_ref[...] - h_ref[...]
    pre = agg * dinv + b_ref[...]
    o_ref[...] = jnp.maximum(pre, 0.0) * dinv


def _tc_out_body(a0_ref, a1_ref, h_ref, dp_ref, w_ref, b_ref, o_ref):
    dinv = _dinv(dp_ref)
    agg = (a0_ref[...] + a1_ref[...] - h_ref[...]) * dinv
    z = jnp.dot(agg, w_ref[...], preferred_element_type=jnp.float32) + b_ref[...]
    m = jnp.max(z, axis=1, keepdims=True)
    lse = jnp.log(jnp.sum(jnp.exp(z - m), axis=1, keepdims=True))
    o_ref[...] = z - m - lse


_tc_in = pl.pallas_call(
    _tc_in_body,
    grid=(N // BLK,),
    in_specs=[
        pl.BlockSpec((BLK, IN_DIM), lambda i: (i, 0)),
        pl.BlockSpec((IN_DIM, HID), lambda i: (0, 0)),
        pl.BlockSpec((BLK, 2), lambda i: (i, 0)),
    ],
    out_specs=pl.BlockSpec((BLK, HID), lambda i: (i, 0)),
    out_shape=jax.ShapeDtypeStruct((N, HID), jnp.float32),
)

_tc_mid = pl.pallas_call(
    _tc_mid_body,
    grid=(N // BLK,),
    in_specs=[
        pl.BlockSpec((BLK, HID), lambda i: (i, 0)),
        pl.BlockSpec((BLK, HID), lambda i: (i, 0)),
        pl.BlockSpec((BLK, HID), lambda i: (i, 0)),
        pl.BlockSpec((BLK, 2), lambda i: (i, 0)),
        pl.BlockSpec((1, HID), lambda i: (0, 0)),
    ],
    out_specs=pl.BlockSpec((BLK, HID), lambda i: (i, 0)),
    out_shape=jax.ShapeDtypeStruct((N, HID), jnp.float32),
)

_tc_out = pl.pallas_call(
    _tc_out_body,
    grid=(N // BLK,),
    in_specs=[
        pl.BlockSpec((BLK, HID), lambda i: (i, 0)),
        pl.BlockSpec((BLK, HID), lambda i: (i, 0)),
        pl.BlockSpec((BLK, HID), lambda i: (i, 0)),
        pl.BlockSpec((BLK, 2), lambda i: (i, 0)),
        pl.BlockSpec((HID, OUT_DIM), lambda i: (0, 0)),
        pl.BlockSpec((1, OUT_DIM), lambda i: (0, 0)),
    ],
    out_specs=pl.BlockSpec((BLK, OUT_DIM), lambda i: (i, 0)),
    out_shape=jax.ShapeDtypeStruct((N, OUT_DIM), jnp.float32),
)


@jax.jit
def kernel(x, edge_index, W1, b1, W2, b2):
    src = edge_index[0].reshape(E // CHUNK, CHUNK)
    dst = edge_index[1].reshape(E // CHUNK, CHUNK)

    degp = _deg_partials(dst).reshape(NC, N)
    dp = degp.T
    h1 = _tc_in(x, W1, dp)
    accp1 = _agg_partials(h1, src, dst)
    h2 = _tc_mid(accp1[0], accp1[1], h1, dp, b1.reshape(1, HID))
    accp2 = _agg_partials(h2, src, dst)
    return _tc_out(accp2[0], accp2[1], h2, dp, W2, b2.reshape(1, OUT_DIM))
